# baseline (speedup 1.0000x reference)
import jax
import jax.numpy as jnp
from jax.experimental import pallas as pl
from jax.experimental.pallas import tpu as pltpu

B, N, IN_C, HID, OUT_C = 16, 1024, 32, 32, 32
MTOT = float(B * N)
EPS = 1e-5


def _body(x_ref, adj_ref, wr1, br1, wl1, g1, be1, wr2, br2, wl2, g2, be2,
          wr3, br3, wl3, g3, be3, wlin, blin, out_ref,
          adjc, r_ref, s1, s2, a_ref, bb_ref):
    t = pl.program_id(0)
    bf16 = jnp.bfloat16

    @pl.when(t == 0)
    def _init_stats():
        s1[...] = jnp.zeros_like(s1)
        s2[...] = jnp.zeros_like(s2)

    @pl.when(jnp.logical_and(t >= 8, t <= 10))
    def _finalize_stats():
        j = t - 8
        g = jnp.where(t == 8, g1[...], jnp.where(t == 9, g2[...], g3[...]))
        be = jnp.where(t == 8, be1[...], jnp.where(t == 9, be2[...], be3[...]))
        mu = s1[...] / MTOT
        var = s2[...] / MTOT - mu * mu
        a = g * jax.lax.rsqrt(var + EPS)
        a_ref[j] = a
        bb_ref[j] = be - mu * a
        s1[...] = jnp.zeros_like(s1)
        s2[...] = jnp.zeros_like(s2)

    def layer(xin_bf, agg, wr, brv, wl, jout, b):
        conv = (jnp.dot(agg.astype(bf16), wr[...].astype(bf16),
                        preferred_element_type=jnp.float32)
                + jnp.dot(xin_bf, wl[...].astype(bf16),
                          preferred_element_type=jnp.float32)
                + brv[...])
        r = jnp.maximum(conv, 0.0)
        y = r.astype(bf16)
        r_ref[jout, b] = y
        yp = jnp.concatenate([y, jnp.ones((N, 1), bf16)], axis=1)
        st = jax.lax.dot_general(yp, yp, (((0,), (0,)), ((), ())),
                                 preferred_element_type=jnp.float32)
        eye = (jax.lax.broadcasted_iota(jnp.int32, (HID, HID), 0)
               == jax.lax.broadcasted_iota(jnp.int32, (HID, HID), 1))
        s1[...] += st[HID:HID + 1, 0:HID]
        s2[...] += jnp.sum(jnp.where(eye, st[0:HID, 0:HID], 0.0),
                           axis=0, keepdims=True)

    def bn_apply(j, b):
        xf = a_ref[j] * r_ref[j, b].astype(jnp.float32) + bb_ref[j]
        return xf.astype(bf16)

    def agg_cached(b, x_bf):
        return jnp.dot(adjc[b], x_bf, preferred_element_type=jnp.float32)

    @pl.when(t < 8)
    def _phase0():
        def body0(i, carry):
            b = 2 * t + i
            ab = adj_ref[i].astype(bf16)
            adjc[b] = ab
            xb = x_ref[i]
            agg = jnp.dot(ab, xb, preferred_element_type=jnp.float32)
            layer(xb, agg, wr1, br1, wl1, 0, b)
            return carry
        jax.lax.fori_loop(0, 2, body0, 0)

    @pl.when(t == 8)
    def _phase1():
        def body1(b, carry):
            x1 = bn_apply(0, b)
            layer(x1, agg_cached(b, x1), wr2, br2, wl2, 1, b)
            return carry
        jax.lax.fori_loop(0, B, body1, 0)

    @pl.when(t == 9)
    def _phase2():
        def body2(b, carry):
            x2 = bn_apply(1, b)
            layer(x2, agg_cached(b, x2), wr3, br3, wl3, 2, b)
            return carry
        jax.lax.fori_loop(0, B, body2, 0)

    @pl.when(t >= 10)
    def _phase3():
        def body3(i, carry):
            b = 2 * (t - 10) + i
            o = (jnp.dot(bn_apply(0, b), wlin[0:HID].astype(bf16),
                         preferred_element_type=jnp.float32)
                 + jnp.dot(bn_apply(1, b), wlin[HID:2 * HID].astype(bf16),
                           preferred_element_type=jnp.float32)
                 + jnp.dot(bn_apply(2, b), wlin[2 * HID:].astype(bf16),
                           preferred_element_type=jnp.float32)
                 + blin[...])
            out_ref[i] = jnp.maximum(o, 0.0)
            return carry
        jax.lax.fori_loop(0, 2, body3, 0)


def kernel(x, adj, Wr1, br1, Wl1, g1, be1, Wr2, br2, Wl2, g2, be2,
           Wr3, br3, Wl3, g3, be3, Wlin, blin):
    vec = lambda v: v.reshape(1, -1)

    def full(arr):
        nd = arr.ndim
        return pl.BlockSpec(arr.shape, lambda t: (0,) * nd)

    small = [vec(br1), Wl1, vec(g1), vec(be1),
             Wr2, vec(br2), Wl2, vec(g2), vec(be2),
             Wr3, vec(br3), Wl3, vec(g3), vec(be3),
             Wlin, vec(blin)]

    in_specs = (
        [pl.BlockSpec((2, N, IN_C),
                      lambda t: (jnp.where(t < 8, t, B // 2 - 1), 0, 0)),
         pl.BlockSpec((2, N, N),
                      lambda t: (jnp.where(t < 8, t, B // 2 - 1), 0, 0)),
         full(Wr1)]
        + [full(a) for a in small]
    )

    return pl.pallas_call(
        _body,
        grid=(18,),
        in_specs=in_specs,
        out_specs=pl.BlockSpec(
            (2, N, OUT_C), lambda t: (jnp.where(t >= 10, t - 10, 0), 0, 0)),
        out_shape=jax.ShapeDtypeStruct((B, N, OUT_C), jnp.float32),
        scratch_shapes=[
            pltpu.VMEM((B, N, N), jnp.bfloat16),
            pltpu.VMEM((3, B, N, HID), jnp.bfloat16),
            pltpu.VMEM((1, HID), jnp.float32),
            pltpu.VMEM((1, HID), jnp.float32),
            pltpu.VMEM((3, 1, HID), jnp.float32),
            pltpu.VMEM((3, 1, HID), jnp.float32),
        ],
        compiler_params=pltpu.CompilerParams(
            dimension_semantics=("arbitrary",),
            vmem_limit_bytes=112 * 1024 * 1024,
        ),
    )(x.astype(jnp.bfloat16), adj, Wr1, *small)


# speedup vs baseline: 1.0972x; 1.0972x over previous
import jax
import jax.numpy as jnp
from jax.experimental import pallas as pl
from jax.experimental.pallas import tpu as pltpu

B, N, IN_C, HID, OUT_C = 16, 1024, 32, 32, 32
MTOT = float(B * N)
EPS = 1e-5


def _body(x_ref, adj_ref, wr1, br1, wl1, g1, be1, wr2, br2, wl2, g2, be2,
          wr3, br3, wl3, g3, be3, wlin, blin, out_ref,
          adjc, r_ref, s1, s2, a_ref, bb_ref):
    t = pl.program_id(0)
    bf16 = jnp.bfloat16

    @pl.when(t == 0)
    def _init_stats():
        s1[...] = jnp.zeros_like(s1)
        s2[...] = jnp.zeros_like(s2)

    @pl.when(jnp.logical_and(t >= 8, t <= 10))
    def _finalize_stats():
        j = t - 8
        g = jnp.where(t == 8, g1[...], jnp.where(t == 9, g2[...], g3[...]))
        be = jnp.where(t == 8, be1[...], jnp.where(t == 9, be2[...], be3[...]))
        mu = s1[...] / MTOT
        var = s2[...] / MTOT - mu * mu
        a = g * jax.lax.rsqrt(var + EPS)
        a_ref[j] = a
        bb_ref[j] = be - mu * a
        s1[...] = jnp.zeros_like(s1)
        s2[...] = jnp.zeros_like(s2)

    def layer(xin_bf, agg, wr, brv, wl, jout, b):
        conv = (jnp.dot(agg.astype(bf16), wr[...].astype(bf16),
                        preferred_element_type=jnp.float32)
                + jnp.dot(xin_bf, wl[...].astype(bf16),
                          preferred_element_type=jnp.float32)
                + brv[...])
        r = jnp.maximum(conv, 0.0)
        r_ref[jout, b] = r.astype(bf16)
        s1[...] += jnp.sum(r, axis=0, keepdims=True)
        s2[...] += jnp.sum(r * r, axis=0, keepdims=True)

    def bn_apply(j, b):
        xf = a_ref[j] * r_ref[j, b].astype(jnp.float32) + bb_ref[j]
        return xf.astype(bf16)

    def agg_cached(b, x_bf):
        return jnp.dot(adjc[b], x_bf, preferred_element_type=jnp.float32)

    @pl.when(t < 8)
    def _phase0():
        def body0(i, carry):
            b = 2 * t + i
            ab = adj_ref[i].astype(bf16)
            adjc[b] = ab
            xb = x_ref[i]
            agg = jnp.dot(ab, xb, preferred_element_type=jnp.float32)
            layer(xb, agg, wr1, br1, wl1, 0, b)
            return carry
        jax.lax.fori_loop(0, 2, body0, 0)

    @pl.when(t == 8)
    def _phase1():
        def body1(b, carry):
            x1 = bn_apply(0, b)
            layer(x1, agg_cached(b, x1), wr2, br2, wl2, 1, b)
            return carry
        jax.lax.fori_loop(0, B, body1, 0)

    @pl.when(t == 9)
    def _phase2():
        def body2(b, carry):
            x2 = bn_apply(1, b)
            layer(x2, agg_cached(b, x2), wr3, br3, wl3, 2, b)
            return carry
        jax.lax.fori_loop(0, B, body2, 0)

    @pl.when(t >= 10)
    def _phase3():
        def body3(i, carry):
            b = 2 * (t - 10) + i
            o = (jnp.dot(bn_apply(0, b), wlin[0:HID].astype(bf16),
                         preferred_element_type=jnp.float32)
                 + jnp.dot(bn_apply(1, b), wlin[HID:2 * HID].astype(bf16),
                           preferred_element_type=jnp.float32)
                 + jnp.dot(bn_apply(2, b), wlin[2 * HID:].astype(bf16),
                           preferred_element_type=jnp.float32)
                 + blin[...])
            out_ref[i, :, 0:OUT_C] = jnp.maximum(o, 0.0)
            return carry
        jax.lax.fori_loop(0, 2, body3, 0)


def kernel(x, adj, Wr1, br1, Wl1, g1, be1, Wr2, br2, Wl2, g2, be2,
           Wr3, br3, Wl3, g3, be3, Wlin, blin):
    vec = lambda v: v.reshape(1, -1)

    def full(arr):
        nd = arr.ndim
        return pl.BlockSpec(arr.shape, lambda t: (0,) * nd)

    small = [vec(br1), Wl1, vec(g1), vec(be1),
             Wr2, vec(br2), Wl2, vec(g2), vec(be2),
             Wr3, vec(br3), Wl3, vec(g3), vec(be3),
             Wlin, vec(blin)]

    in_specs = (
        [pl.BlockSpec((2, N, IN_C),
                      lambda t: (jnp.where(t < 8, t, B // 2 - 1), 0, 0)),
         pl.BlockSpec((2, N, N),
                      lambda t: (jnp.where(t < 8, t, B // 2 - 1), 0, 0)),
         full(Wr1)]
        + [full(a) for a in small]
    )

    res = pl.pallas_call(
        _body,
        grid=(18,),
        in_specs=in_specs,
        out_specs=pl.BlockSpec(
            (2, N, 128), lambda t: (jnp.where(t >= 10, t - 10, 0), 0, 0)),
        out_shape=jax.ShapeDtypeStruct((B, N, 128), jnp.float32),
        scratch_shapes=[
            pltpu.VMEM((B, N, N), jnp.bfloat16),
            pltpu.VMEM((3, B, N, HID), jnp.bfloat16),
            pltpu.VMEM((1, HID), jnp.float32),
            pltpu.VMEM((1, HID), jnp.float32),
            pltpu.VMEM((3, 1, HID), jnp.float32),
            pltpu.VMEM((3, 1, HID), jnp.float32),
        ],
        compiler_params=pltpu.CompilerParams(
            dimension_semantics=("arbitrary",),
            vmem_limit_bytes=112 * 1024 * 1024,
        ),
    )(x.astype(jnp.bfloat16), adj, Wr1, *small)
    return res[:, :, :OUT_C]


# speedup vs baseline: 1.0981x; 1.0009x over previous
import jax
import jax.numpy as jnp
from jax.experimental import pallas as pl
from jax.experimental.pallas import tpu as pltpu

B, N, IN_C, HID, OUT_C = 16, 1024, 32, 32, 32
MTOT = float(B * N)
EPS = 1e-5


def _body(x_ref, adj_ref, wr1, br1, wl1, g1, be1, wr2, br2, wl2, g2, be2,
          wr3, br3, wl3, g3, be3, wlin, blin, out_ref,
          adjc, r_ref, s1, s2, a_ref, bb_ref):
    t = pl.program_id(0)
    bf16 = jnp.bfloat16

    @pl.when(t == 0)
    def _init_stats():
        s1[...] = jnp.zeros_like(s1)
        s2[...] = jnp.zeros_like(s2)

    def finalize(j, g, be):
        mu = s1[...] / MTOT
        var = s2[...] / MTOT - mu * mu
        a = g[...] * jax.lax.rsqrt(var + EPS)
        a_ref[j] = a
        bb_ref[j] = be[...] - mu * a
        s1[...] = jnp.zeros_like(s1)
        s2[...] = jnp.zeros_like(s2)

    def layer(xin_bf, agg, wr, brv, wl, jout, b):
        conv = (jnp.dot(agg.astype(bf16), wr[...].astype(bf16),
                        preferred_element_type=jnp.float32)
                + jnp.dot(xin_bf, wl[...].astype(bf16),
                          preferred_element_type=jnp.float32)
                + brv[...])
        r = jnp.maximum(conv, 0.0)
        r_ref[jout, b] = r.astype(bf16)
        s1[...] += jnp.sum(r, axis=0, keepdims=True)
        s2[...] += jnp.sum(r * r, axis=0, keepdims=True)

    def bn_apply(j, b):
        xf = a_ref[j] * r_ref[j, b].astype(jnp.float32) + bb_ref[j]
        return xf.astype(bf16)

    def agg_cached(b, x_bf):
        return jnp.dot(adjc[b], x_bf, preferred_element_type=jnp.float32)

    @pl.when(t < 8)
    def _phase0():
        def body0(i, carry):
            b = 2 * t + i
            ab = adj_ref[i].astype(bf16)
            adjc[b] = ab
            xb = x_ref[i]
            agg = jnp.dot(ab, xb, preferred_element_type=jnp.float32)
            layer(xb, agg, wr1, br1, wl1, 0, b)
            return carry
        jax.lax.fori_loop(0, 2, body0, 0)

    @pl.when(t == 8)
    def _phase12():
        finalize(0, g1, be1)

        def body1(b, carry):
            x1 = bn_apply(0, b)
            layer(x1, agg_cached(b, x1), wr2, br2, wl2, 1, b)
            return carry
        jax.lax.fori_loop(0, B, body1, 0)
        finalize(1, g2, be2)

        def body2(b, carry):
            x2 = bn_apply(1, b)
            layer(x2, agg_cached(b, x2), wr3, br3, wl3, 2, b)
            return carry
        jax.lax.fori_loop(0, B, body2, 0)

    @pl.when(t == 9)
    def _finalize3():
        finalize(2, g3, be3)

    @pl.when(t >= 9)
    def _phase3():
        def body3(i, carry):
            b = 2 * (t - 9) + i
            o = (jnp.dot(bn_apply(0, b), wlin[0:HID].astype(bf16),
                         preferred_element_type=jnp.float32)
                 + jnp.dot(bn_apply(1, b), wlin[HID:2 * HID].astype(bf16),
                           preferred_element_type=jnp.float32)
                 + jnp.dot(bn_apply(2, b), wlin[2 * HID:].astype(bf16),
                           preferred_element_type=jnp.float32)
                 + blin[...])
            out_ref[i, :, 0:OUT_C] = jnp.maximum(o, 0.0)
            return carry
        jax.lax.fori_loop(0, 2, body3, 0)


def kernel(x, adj, Wr1, br1, Wl1, g1, be1, Wr2, br2, Wl2, g2, be2,
           Wr3, br3, Wl3, g3, be3, Wlin, blin):
    vec = lambda v: v.reshape(1, -1)

    def full(arr):
        nd = arr.ndim
        return pl.BlockSpec(arr.shape, lambda t: (0,) * nd)

    small = [vec(br1), Wl1, vec(g1), vec(be1),
             Wr2, vec(br2), Wl2, vec(g2), vec(be2),
             Wr3, vec(br3), Wl3, vec(g3), vec(be3),
             Wlin, vec(blin)]

    in_specs = (
        [pl.BlockSpec((2, N, IN_C),
                      lambda t: (jnp.where(t < 8, t, B // 2 - 1), 0, 0)),
         pl.BlockSpec((2, N, N),
                      lambda t: (jnp.where(t < 8, t, B // 2 - 1), 0, 0)),
         full(Wr1)]
        + [full(a) for a in small]
    )

    res = pl.pallas_call(
        _body,
        grid=(17,),
        in_specs=in_specs,
        out_specs=pl.BlockSpec(
            (2, N, 128), lambda t: (jnp.where(t >= 9, t - 9, 0), 0, 0)),
        out_shape=jax.ShapeDtypeStruct((B, N, 128), jnp.float32),
        scratch_shapes=[
            pltpu.VMEM((B, N, N), jnp.bfloat16),
            pltpu.VMEM((3, B, N, HID), jnp.bfloat16),
            pltpu.VMEM((1, HID), jnp.float32),
            pltpu.VMEM((1, HID), jnp.float32),
            pltpu.VMEM((3, 1, HID), jnp.float32),
            pltpu.VMEM((3, 1, HID), jnp.float32),
        ],
        compiler_params=pltpu.CompilerParams(
            dimension_semantics=("arbitrary",),
            vmem_limit_bytes=112 * 1024 * 1024,
        ),
    )(x.astype(jnp.bfloat16), adj, Wr1, *small)
    return res[:, :, :OUT_C]


# speedup vs baseline: 1.1019x; 1.0035x over previous
import jax
import jax.numpy as jnp
from jax.experimental import pallas as pl
from jax.experimental.pallas import tpu as pltpu

B, N, IN_C, HID, OUT_C = 16, 1024, 32, 32, 32
MTOT = float(B * N)
EPS = 1e-5


def _body(x_ref, adj_ref, wr1, br1, wl1, g1, be1, wr2, br2, wl2, g2, be2,
          wr3, br3, wl3, g3, be3, wlin, blin, out_ref,
          adjc, r_ref, s1, s2, a_ref, bb_ref, wle_ref, bc_ref):
    t = pl.program_id(0)
    bf16 = jnp.bfloat16

    @pl.when(t == 0)
    def _init_stats():
        s1[...] = jnp.zeros_like(s1)
        s2[...] = jnp.zeros_like(s2)

    def finalize(j, g, be):
        mu = s1[...] / MTOT
        var = s2[...] / MTOT - mu * mu
        a = g[...] * jax.lax.rsqrt(var + EPS)
        a_ref[j] = a
        bb_ref[j] = be[...] - mu * a
        s1[...] = jnp.zeros_like(s1)
        s2[...] = jnp.zeros_like(s2)

    def layer(xin_bf, agg, wr, brv, wl, jout, b):
        conv = (jnp.dot(agg.astype(bf16), wr[...].astype(bf16),
                        preferred_element_type=jnp.float32)
                + jnp.dot(xin_bf, wl[...].astype(bf16),
                          preferred_element_type=jnp.float32)
                + brv[...])
        r = jnp.maximum(conv, 0.0)
        r_ref[jout, b] = r.astype(bf16)
        s1[...] += jnp.sum(r, axis=0, keepdims=True)
        s2[...] += jnp.sum(r * r, axis=0, keepdims=True)

    def bn_apply(j, b):
        xf = a_ref[j] * r_ref[j, b].astype(jnp.float32) + bb_ref[j]
        return xf.astype(bf16)

    def agg_cached(b, x_bf):
        return jnp.dot(adjc[b], x_bf, preferred_element_type=jnp.float32)

    @pl.when(t < 8)
    def _phase0():
        def body0(i, carry):
            b = 2 * t + i
            ab = adj_ref[i].astype(bf16)
            adjc[b] = ab
            xb = x_ref[i]
            agg = jnp.dot(ab, xb, preferred_element_type=jnp.float32)
            layer(xb, agg, wr1, br1, wl1, 0, b)
            return carry
        jax.lax.fori_loop(0, 2, body0, 0)

    @pl.when(t == 8)
    def _phase12():
        finalize(0, g1, be1)

        def body1(b, carry):
            x1 = bn_apply(0, b)
            layer(x1, agg_cached(b, x1), wr2, br2, wl2, 1, b)
            return carry
        jax.lax.fori_loop(0, B, body1, 0)
        finalize(1, g2, be2)

        def body2(b, carry):
            x2 = bn_apply(1, b)
            layer(x2, agg_cached(b, x2), wr3, br3, wl3, 2, b)
            return carry
        jax.lax.fori_loop(0, B, body2, 0)

    @pl.when(t == 9)
    def _finalize3():
        finalize(2, g3, be3)
        bc = blin[...]
        for j in range(3):
            wj = wlin[HID * j:HID * (j + 1)]
            wle_ref[j] = (wj * a_ref[j].reshape(HID, 1)).astype(bf16)
            bc = bc + jnp.dot(bb_ref[j], wj,
                              preferred_element_type=jnp.float32)
        bc_ref[...] = bc

    @pl.when(t >= 9)
    def _phase3():
        def body3(i, carry):
            b = 2 * (t - 9) + i
            o = (jnp.dot(r_ref[0, b], wle_ref[0],
                         preferred_element_type=jnp.float32)
                 + jnp.dot(r_ref[1, b], wle_ref[1],
                           preferred_element_type=jnp.float32)
                 + jnp.dot(r_ref[2, b], wle_ref[2],
                           preferred_element_type=jnp.float32)
                 + bc_ref[...])
            out_ref[i, :, 0:OUT_C] = jnp.maximum(o, 0.0)
            return carry
        jax.lax.fori_loop(0, 2, body3, 0)


def kernel(x, adj, Wr1, br1, Wl1, g1, be1, Wr2, br2, Wl2, g2, be2,
           Wr3, br3, Wl3, g3, be3, Wlin, blin):
    vec = lambda v: v.reshape(1, -1)

    def full(arr):
        nd = arr.ndim
        return pl.BlockSpec(arr.shape, lambda t: (0,) * nd)

    small = [vec(br1), Wl1, vec(g1), vec(be1),
             Wr2, vec(br2), Wl2, vec(g2), vec(be2),
             Wr3, vec(br3), Wl3, vec(g3), vec(be3),
             Wlin, vec(blin)]

    in_specs = (
        [pl.BlockSpec((2, N, IN_C),
                      lambda t: (jnp.where(t < 8, t, B // 2 - 1), 0, 0)),
         pl.BlockSpec((2, N, N),
                      lambda t: (jnp.where(t < 8, t, B // 2 - 1), 0, 0)),
         full(Wr1)]
        + [full(a) for a in small]
    )

    res = pl.pallas_call(
        _body,
        grid=(17,),
        in_specs=in_specs,
        out_specs=pl.BlockSpec(
            (2, N, 128), lambda t: (jnp.where(t >= 9, t - 9, 0), 0, 0)),
        out_shape=jax.ShapeDtypeStruct((B, N, 128), jnp.float32),
        scratch_shapes=[
            pltpu.VMEM((B, N, N), jnp.bfloat16),
            pltpu.VMEM((3, B, N, HID), jnp.bfloat16),
            pltpu.VMEM((1, HID), jnp.float32),
            pltpu.VMEM((1, HID), jnp.float32),
            pltpu.VMEM((3, 1, HID), jnp.float32),
            pltpu.VMEM((3, 1, HID), jnp.float32),
            pltpu.VMEM((3, HID, HID), jnp.bfloat16),
            pltpu.VMEM((1, HID), jnp.float32),
        ],
        compiler_params=pltpu.CompilerParams(
            dimension_semantics=("arbitrary",),
            vmem_limit_bytes=112 * 1024 * 1024,
        ),
    )(x.astype(jnp.bfloat16), adj, Wr1, *small)
    return res[:, :, :OUT_C]
